# baseline (speedup 1.0000x reference)
import dataclasses
import functools

import jax
import jax.numpy as jnp
from jax import lax
from jax.experimental import pallas as pl
from jax.experimental.pallas import tpu as pltpu
from jax.experimental.pallas import tpu_sc as plsc

NC = 2
NS = 16
L = 16
NW = NC * NS


def _zero_rows(zbuf, acc, base, rows, zrows):
    full = (rows // zrows) * zrows

    @pl.loop(0, full, step=zrows)
    def _(q):
        pltpu.sync_copy(zbuf, acc.at[pl.ds(base + q, zrows)])

    rem = rows - full
    if rem:
        pltpu.sync_copy(zbuf.at[pl.ds(0, rem)], acc.at[pl.ds(base + full, rem)])


def _sc_mesh():
    return plsc.VectorSubcoreMesh(
        core_axis_name="c", subcore_axis_name="s", num_cores=NC, num_subcores=NS
    )


def _sc_params():
    cp = pltpu.CompilerParams()
    if "needs_layout_passes" in pltpu.CompilerParams.__dataclass_fields__:
        cp = dataclasses.replace(cp, needs_layout_passes=False)
    return cp


def _sc_degree(col3, ew3, n):
    nchunk, c = col3.shape[1], col3.shape[2]

    @functools.partial(
        pl.kernel,
        out_type=jax.ShapeDtypeStruct((NW, n), jnp.float32),
        mesh=_sc_mesh(),
        compiler_params=_sc_params(),
        scratch_types=[
            pltpu.VMEM((n,), jnp.float32),
            pltpu.VMEM((nchunk, c), jnp.int32),
            pltpu.VMEM((nchunk, c), jnp.float32),
        ],
    )
    def deg_kernel(col_hbm, ew_hbm, out_hbm, degv, col_v, ew_v):
        cid = lax.axis_index("c")
        sid = lax.axis_index("s")
        wid = sid * NC + cid

        zero = jnp.zeros((L,), jnp.float32)

        @pl.loop(0, n, step=L)
        def _(r):
            degv[pl.ds(r, L)] = zero

        pltpu.sync_copy(col_hbm.at[wid], col_v)
        pltpu.sync_copy(ew_hbm.at[wid], ew_v)

        @pl.loop(0, nchunk)
        def _(j):
            @pl.loop(0, c, step=L)
            def _(e0):
                cv = col_v[j, pl.ds(e0, L)]
                wv = ew_v[j, pl.ds(e0, L)]
                plsc.addupdate_scatter(degv, [cv], wv)

        pltpu.sync_copy(degv, out_hbm.at[wid])

    return deg_kernel(col3, ew3)


def _sc_edge_pass(y, row4, col4, ew4, n, d, d_scale):
    ngroups, G, c = row4.shape[1], row4.shape[2], row4.shape[3]
    rpt = n // NS

    @functools.partial(
        pl.kernel,
        out_type=jax.ShapeDtypeStruct((NC, NS, n // NS, d), jnp.float32),
        mesh=_sc_mesh(),
        compiler_params=_sc_params(),
        scratch_types=[
            pltpu.VMEM_SHARED((n, d), jnp.float32),
            pltpu.VMEM((G, c), jnp.int32),
            pltpu.VMEM((G, c), jnp.int32),
            pltpu.VMEM((G, c), jnp.float32),
            pltpu.VMEM((c, d), jnp.float32),
            pltpu.VMEM((c, d), jnp.float32),
            pltpu.VMEM((c, d), jnp.float32),
            pltpu.SemaphoreType.DMA,
            pltpu.SemaphoreType.DMA,
            pltpu.SemaphoreType.DMA,
            pltpu.SemaphoreType.DMA,
            pltpu.SemaphoreType.DMA,
            pltpu.SemaphoreType.DMA,
        ],
    )
    def edge_kernel(
        y_hbm, row_hbm, col_hbm, ew_hbm, out_hbm,
        acc, row_v, col_v, ew_v, b0, b1, b2,
        gsem0, gsem1, gsem2, ssem0, ssem1, ssem2,
    ):
        c2 = c // 2
        cid = lax.axis_index("c")
        sid = lax.axis_index("s")
        wid = sid * NC + cid
        bufs = (b0, b1, b2)
        gsems = (gsem0, gsem1, gsem2)
        ssems = (ssem0, ssem1, ssem2)

        zero = jnp.zeros((L,), jnp.float32)

        @pl.loop(0, c)
        def _(r):
            for t in range(d // L):
                b0[r, pl.ds(t * L, L)] = zero

        _zero_rows(b0, acc, sid * rpt, rpt, c)

        plsc.subcore_barrier()

        def scale(buf, j):
            @plsc.parallel_loop(0, c, L, unroll=2)
            def _(e0):
                wv = ew_v[j, pl.ds(e0, L)]
                for k in range(L):
                    w = wv[jnp.full((L,), k, jnp.int32)]
                    for t in range(d_scale // L):
                        sl = pl.ds(t * L, L)
                        buf[e0 + k, sl] = buf[e0 + k, sl] * w

        @pl.loop(0, ngroups)
        def _(g):
            pltpu.sync_copy(row_hbm.at[wid, g], row_v)
            pltpu.sync_copy(col_hbm.at[wid, g], col_v)
            pltpu.sync_copy(ew_hbm.at[wid, g], ew_v)

            pltpu.async_copy(y_hbm.at[row_v.at[0, pl.ds(0, c2)]], b0.at[pl.ds(0, c2)], gsem0)
            pltpu.async_copy(y_hbm.at[row_v.at[0, pl.ds(c2, c2)]], b0.at[pl.ds(c2, c2)], gsem0)

            @pl.loop(0, G, step=3)
            def _(i0):
                for b in range(3):
                    i = i0 + b
                    nb = (b + 1) % 3
                    pltpu.make_async_copy(
                        y_hbm.at[row_v.at[i]], bufs[b], gsems[b]
                    ).wait()

                    @pl.when(i + 1 < G)
                    def _():
                        @pl.when(i >= 2)
                        def _():
                            pltpu.make_async_copy(
                                bufs[nb], acc.at[col_v.at[i - 2]], ssems[nb]
                            ).wait()

                        pltpu.async_copy(
                            y_hbm.at[row_v.at[i + 1, pl.ds(0, c2)]],
                            bufs[nb].at[pl.ds(0, c2)], gsems[nb],
                        )
                        pltpu.async_copy(
                            y_hbm.at[row_v.at[i + 1, pl.ds(c2, c2)]],
                            bufs[nb].at[pl.ds(c2, c2)], gsems[nb],
                        )

                    scale(bufs[b], i)
                    pltpu.async_copy(
                        bufs[b], acc.at[col_v.at[i]], ssems[b], add=True
                    )

            for b in range(3):
                i = G - 3 + b
                pltpu.make_async_copy(
                    bufs[b], acc.at[col_v.at[i]], ssems[b]
                ).wait()

        plsc.subcore_barrier()
        pltpu.sync_copy(acc.at[pl.ds(sid * rpt, rpt)], out_hbm.at[cid, sid])

    return edge_kernel(y, row4, col4, ew4)


def _tc_scale(x, w, dp, br):
    n, din = x.shape
    dout = w.shape[0]
    grid = n // br

    def body(x_ref, w_ref, dp_ref, y_ref):
        xw = lax.dot_general(
            x_ref[...], w_ref[...], (((1,), (1,)), ((), ())),
            preferred_element_type=jnp.float32,
        )
        deg = 1.0 + jnp.sum(dp_ref[...], axis=1, keepdims=True)
        y_ref[...] = xw * lax.rsqrt(deg)

    return pl.pallas_call(
        body,
        grid=(grid,),
        in_specs=[
            pl.BlockSpec((br, din), lambda i: (i, 0)),
            pl.BlockSpec((dout, din), lambda i: (0, 0)),
            pl.BlockSpec((br, NC * L), lambda i: (i, 0)),
        ],
        out_specs=pl.BlockSpec((br, dout), lambda i: (i, 0)),
        out_shape=jax.ShapeDtypeStruct((n, dout), jnp.float32),
    )(x, w, dp)


def _tc_combine_matmul(p1, y1, dp, w2, b1, br):
    n, d = y1.shape
    dout = w2.shape[0]
    grid = n // br

    def body(p_ref, y_ref, dp_ref, w_ref, b_ref, o_ref):
        deg = 1.0 + jnp.sum(dp_ref[...], axis=1, keepdims=True)
        dis = lax.rsqrt(deg)
        h = dis * (p_ref[0] + p_ref[1] + y_ref[...]) + b_ref[...]
        h = jnp.maximum(h, 0.0)
        xw = lax.dot_general(
            h, w_ref[...], (((1,), (1,)), ((), ())),
            preferred_element_type=jnp.float32,
        )
        o_ref[...] = xw * dis

    return pl.pallas_call(
        body,
        grid=(grid,),
        in_specs=[
            pl.BlockSpec((NC, br, d), lambda i: (0, i, 0)),
            pl.BlockSpec((br, d), lambda i: (i, 0)),
            pl.BlockSpec((br, NC * L), lambda i: (i, 0)),
            pl.BlockSpec((dout, d), lambda i: (0, 0)),
            pl.BlockSpec((1, d), lambda i: (0, 0)),
        ],
        out_specs=pl.BlockSpec((br, dout), lambda i: (i, 0)),
        out_shape=jax.ShapeDtypeStruct((n, dout), jnp.float32),
    )(p1, y1, dp, w2, b1)


def _tc_final(p2, y2, dp, b2, br, d_out):
    n, d = y2.shape
    grid = n // br

    def body(p_ref, y_ref, dp_ref, b_ref, o_ref):
        deg = 1.0 + jnp.sum(dp_ref[...], axis=1, keepdims=True)
        dis = lax.rsqrt(deg)
        v = dis * (p_ref[0] + p_ref[1] + y_ref[...])
        o_ref[...] = v[:, :d_out] + b_ref[...]

    return pl.pallas_call(
        body,
        grid=(grid,),
        in_specs=[
            pl.BlockSpec((NC, br, d), lambda i: (0, i, 0)),
            pl.BlockSpec((br, d), lambda i: (i, 0)),
            pl.BlockSpec((br, NC * L), lambda i: (i, 0)),
            pl.BlockSpec((1, d_out), lambda i: (0, 0)),
        ],
        out_specs=pl.BlockSpec((br, d_out), lambda i: (i, 0)),
        out_shape=jax.ShapeDtypeStruct((n, d_out), jnp.float32),
    )(p2, y2, dp, b2)


def kernel(x, edge_index, edge_weight, W1, b1, W2, b2):
    n, d_in = x.shape
    e = edge_index.shape[1]
    d_hid = W1.shape[0]
    d_out = W2.shape[0]

    c = 112
    G = 15
    ept = -(-e // NW)
    nchunk = -(-(-(-ept // c)) // G) * G
    ngroups = nchunk // G
    e_pad = NW * nchunk * c

    row = edge_index[0]
    col = edge_index[1]
    ew = edge_weight
    if e_pad != e:
        pad = e_pad - e
        fill = (jnp.arange(pad, dtype=jnp.int32) % n).astype(jnp.int32)
        row = jnp.concatenate([row, fill])
        col = jnp.concatenate([col, fill])
        ew = jnp.concatenate([ew, jnp.zeros((pad,), jnp.float32)])

    col3 = col.reshape(NW, nchunk, c)
    ew3 = ew.reshape(NW, nchunk, c)
    row4 = row.reshape(NW, ngroups, G, c)
    col4 = col.reshape(NW, ngroups, G, c)
    ew4 = ew.reshape(NW, ngroups, G, c)

    br = 400

    dpart = _sc_degree(col3, ew3, n)
    dp = dpart.T

    w2p = jnp.zeros((d_hid, d_hid), jnp.float32).at[:d_out, :].set(W2)

    y1 = _tc_scale(x, W1, dp, br)
    p1 = _sc_edge_pass(y1, row4, col4, ew4, n, d_hid, d_hid).reshape(NC, n, d_hid)
    y2 = _tc_combine_matmul(p1, y1, dp, w2p, b1[None, :], br)
    p2 = _sc_edge_pass(y2, row4, col4, ew4, n, d_hid, d_out).reshape(NC, n, d_hid)
    return _tc_final(p2, y2, dp, b2[None, :], br, d_out)


# speedup vs baseline: 1.1223x; 1.1223x over previous
import dataclasses
import functools

import jax
import jax.numpy as jnp
from jax import lax
from jax.experimental import pallas as pl
from jax.experimental.pallas import tpu as pltpu
from jax.experimental.pallas import tpu_sc as plsc

NC = 2
NS = 16
L = 16
NW = NC * NS


def _zero_rows(zbuf, acc, base, rows, zrows):
    full = (rows // zrows) * zrows

    @pl.loop(0, full, step=zrows)
    def _(q):
        pltpu.sync_copy(zbuf, acc.at[pl.ds(base + q, zrows)])

    rem = rows - full
    if rem:
        pltpu.sync_copy(zbuf.at[pl.ds(0, rem)], acc.at[pl.ds(base + full, rem)])


def _sc_mesh():
    return plsc.VectorSubcoreMesh(
        core_axis_name="c", subcore_axis_name="s", num_cores=NC, num_subcores=NS
    )


def _sc_params():
    cp = pltpu.CompilerParams()
    if "needs_layout_passes" in pltpu.CompilerParams.__dataclass_fields__:
        cp = dataclasses.replace(cp, needs_layout_passes=False)
    return cp


def _sc_degree(col3, ew3, n):
    nchunk, c = col3.shape[1], col3.shape[2]

    @functools.partial(
        pl.kernel,
        out_type=jax.ShapeDtypeStruct((NW, n), jnp.float32),
        mesh=_sc_mesh(),
        compiler_params=_sc_params(),
        scratch_types=[
            pltpu.VMEM((n,), jnp.float32),
            pltpu.VMEM((nchunk, c), jnp.int32),
            pltpu.VMEM((nchunk, c), jnp.float32),
        ],
    )
    def deg_kernel(col_hbm, ew_hbm, out_hbm, degv, col_v, ew_v):
        cid = lax.axis_index("c")
        sid = lax.axis_index("s")
        wid = sid * NC + cid

        zero = jnp.zeros((L,), jnp.float32)

        @pl.loop(0, n, step=L)
        def _(r):
            degv[pl.ds(r, L)] = zero

        pltpu.sync_copy(col_hbm.at[wid], col_v)
        pltpu.sync_copy(ew_hbm.at[wid], ew_v)

        @pl.loop(0, nchunk)
        def _(j):
            @pl.loop(0, c, step=L)
            def _(e0):
                cv = col_v[j, pl.ds(e0, L)]
                wv = ew_v[j, pl.ds(e0, L)]
                plsc.addupdate_scatter(degv, [cv], wv)

        pltpu.sync_copy(degv, out_hbm.at[wid])

    return deg_kernel(col3, ew3)


def _sc_edge_pass(y, row4, col4, ew4, n, d, d_scale):
    ngroups, G, c = row4.shape[1], row4.shape[2], row4.shape[3]
    rb = (n // NS + 7) // 8 * 8
    last = n - (NS - 1) * rb

    @functools.partial(
        pl.kernel,
        out_type=jax.ShapeDtypeStruct((NC, n, d), jnp.float32),
        mesh=_sc_mesh(),
        compiler_params=_sc_params(),
        scratch_types=[
            pltpu.VMEM_SHARED((n, d), jnp.float32),
            pltpu.VMEM((G, c), jnp.int32),
            pltpu.VMEM((G, c), jnp.int32),
            pltpu.VMEM((G, c), jnp.float32),
            pltpu.VMEM((c, d), jnp.float32),
            pltpu.VMEM((c, d), jnp.float32),
            pltpu.VMEM((c, d), jnp.float32),
            pltpu.SemaphoreType.DMA,
            pltpu.SemaphoreType.DMA,
            pltpu.SemaphoreType.DMA,
            pltpu.SemaphoreType.DMA,
            pltpu.SemaphoreType.DMA,
            pltpu.SemaphoreType.DMA,
        ],
    )
    def edge_kernel(
        y_hbm, row_hbm, col_hbm, ew_hbm, out_hbm,
        acc, row_v, col_v, ew_v, b0, b1, b2,
        gsem0, gsem1, gsem2, ssem0, ssem1, ssem2,
    ):
        c2 = c // 2
        cid = lax.axis_index("c")
        sid = lax.axis_index("s")
        wid = sid * NC + cid
        bufs = (b0, b1, b2)
        gsems = (gsem0, gsem1, gsem2)
        ssems = (ssem0, ssem1, ssem2)

        zero = jnp.zeros((L,), jnp.float32)

        @pl.loop(0, c)
        def _(r):
            for t in range(d // L):
                b0[r, pl.ds(t * L, L)] = zero

        @pl.when(sid < NS - 1)
        def _():
            _zero_rows(b0, acc, sid * rb, rb, c)

        @pl.when(sid == NS - 1)
        def _():
            _zero_rows(b0, acc, (NS - 1) * rb, last, c)

        plsc.subcore_barrier()

        def scale(buf, j):
            @plsc.parallel_loop(0, c, L, unroll=2)
            def _(e0):
                wv = ew_v[j, pl.ds(e0, L)]
                for k in range(L):
                    w = wv[jnp.full((L,), k, jnp.int32)]
                    for t in range(d_scale // L):
                        sl = pl.ds(t * L, L)
                        buf[e0 + k, sl] = buf[e0 + k, sl] * w

        @pl.loop(0, ngroups)
        def _(g):
            pltpu.sync_copy(row_hbm.at[wid, g], row_v)
            pltpu.sync_copy(col_hbm.at[wid, g], col_v)
            pltpu.sync_copy(ew_hbm.at[wid, g], ew_v)

            pltpu.async_copy(y_hbm.at[row_v.at[0, pl.ds(0, c2)]], b0.at[pl.ds(0, c2)], gsem0)
            pltpu.async_copy(y_hbm.at[row_v.at[0, pl.ds(c2, c2)]], b0.at[pl.ds(c2, c2)], gsem0)

            @pl.loop(0, G, step=3)
            def _(i0):
                for b in range(3):
                    i = i0 + b
                    nb = (b + 1) % 3
                    pltpu.make_async_copy(
                        y_hbm.at[row_v.at[i]], bufs[b], gsems[b]
                    ).wait()

                    @pl.when(i + 1 < G)
                    def _():
                        @pl.when(i >= 2)
                        def _():
                            pltpu.make_async_copy(
                                bufs[nb], acc.at[col_v.at[i - 2]], ssems[nb]
                            ).wait()

                        pltpu.async_copy(
                            y_hbm.at[row_v.at[i + 1, pl.ds(0, c2)]],
                            bufs[nb].at[pl.ds(0, c2)], gsems[nb],
                        )
                        pltpu.async_copy(
                            y_hbm.at[row_v.at[i + 1, pl.ds(c2, c2)]],
                            bufs[nb].at[pl.ds(c2, c2)], gsems[nb],
                        )

                    scale(bufs[b], i)
                    pltpu.async_copy(
                        bufs[b], acc.at[col_v.at[i]], ssems[b], add=True
                    )

            for b in range(3):
                i = G - 3 + b
                pltpu.make_async_copy(
                    bufs[b], acc.at[col_v.at[i]], ssems[b]
                ).wait()

        plsc.subcore_barrier()

        @pl.when(sid < NS - 1)
        def _():
            pltpu.sync_copy(
                acc.at[pl.ds(sid * rb, rb)],
                out_hbm.at[cid, pl.ds(sid * rb, rb)],
            )

        @pl.when(sid == NS - 1)
        def _():
            pltpu.sync_copy(
                acc.at[pl.ds((NS - 1) * rb, last)],
                out_hbm.at[cid, pl.ds((NS - 1) * rb, last)],
            )

    return edge_kernel(y, row4, col4, ew4)


def _tc_scale(x, w, dp, br):
    n, din = x.shape
    dout = w.shape[0]

    def body(x_ref, w_ref, dp_ref, y_ref):
        xw = lax.dot_general(
            x_ref[...], w_ref[...], (((1,), (1,)), ((), ())),
            preferred_element_type=jnp.float32,
        )
        deg = 1.0 + jnp.sum(dp_ref[...], axis=1, keepdims=True)
        y_ref[...] = xw * lax.rsqrt(deg)

    return pl.pallas_call(
        body,
        out_shape=jax.ShapeDtypeStruct((n, dout), jnp.float32),
    )(x, w, dp)


def _tc_combine_matmul(p1, y1, dp, w2, b1, br):
    n, d = y1.shape
    dout = w2.shape[0]

    def body(p_ref, y_ref, dp_ref, w_ref, b_ref, o_ref):
        deg = 1.0 + jnp.sum(dp_ref[...], axis=1, keepdims=True)
        dis = lax.rsqrt(deg)
        h = dis * (p_ref[0] + p_ref[1] + y_ref[...]) + b_ref[...]
        h = jnp.maximum(h, 0.0)
        xw = lax.dot_general(
            h, w_ref[...], (((1,), (1,)), ((), ())),
            preferred_element_type=jnp.float32,
        )
        o_ref[...] = xw * dis

    return pl.pallas_call(
        body,
        out_shape=jax.ShapeDtypeStruct((n, dout), jnp.float32),
    )(p1, y1, dp, w2, b1)


def _tc_final(p2, y2, dp, b2, br, d_out):
    n, d = y2.shape

    def body(p_ref, y_ref, dp_ref, b_ref, o_ref):
        deg = 1.0 + jnp.sum(dp_ref[...], axis=1, keepdims=True)
        dis = lax.rsqrt(deg)
        v = dis * (p_ref[0] + p_ref[1] + y_ref[...])
        o_ref[...] = v[:, :d_out] + b_ref[...]

    return pl.pallas_call(
        body,
        out_shape=jax.ShapeDtypeStruct((n, d_out), jnp.float32),
    )(p2, y2, dp, b2)


def kernel(x, edge_index, edge_weight, W1, b1, W2, b2):
    n, d_in = x.shape
    e = edge_index.shape[1]
    d_hid = W1.shape[0]
    d_out = W2.shape[0]

    c = 112
    G = 15
    ept = -(-e // NW)
    nchunk = -(-(-(-ept // c)) // G) * G
    ngroups = nchunk // G
    e_pad = NW * nchunk * c

    row = edge_index[0]
    col = edge_index[1]
    ew = edge_weight
    if e_pad != e:
        pad = e_pad - e
        fill = (jnp.arange(pad, dtype=jnp.int32) % n).astype(jnp.int32)
        row = jnp.concatenate([row, fill])
        col = jnp.concatenate([col, fill])
        ew = jnp.concatenate([ew, jnp.zeros((pad,), jnp.float32)])

    col3 = col.reshape(NW, nchunk, c)
    ew3 = ew.reshape(NW, nchunk, c)
    row4 = row.reshape(NW, ngroups, G, c)
    col4 = col.reshape(NW, ngroups, G, c)
    ew4 = ew.reshape(NW, ngroups, G, c)

    br = 400

    dpart = _sc_degree(col3, ew3, n)
    dp = dpart.T

    w2p = jnp.zeros((d_hid, d_hid), jnp.float32).at[:d_out, :].set(W2)

    y1 = _tc_scale(x, W1, dp, br)
    p1 = _sc_edge_pass(y1, row4, col4, ew4, n, d_hid, d_hid)
    y2 = _tc_combine_matmul(p1, y1, dp, w2p, b1[None, :], br)
    p2 = _sc_edge_pass(y2, row4, col4, ew4, n, d_hid, d_out)
    return _tc_final(p2, y2, dp, b2[None, :], br, d_out)
